# Initial kernel scaffold; baseline (speedup 1.0000x reference)
import jax
import jax.numpy as jnp
from jax.experimental import pallas as pl


def kernel(s_feat, o_feat, ss_feat, os_feat, Ws_w, Ws_b, Wos_w, Wos_b, Wss_w, Wss_b, attn_w, attn_b, Win_w, Win_b, Wself_w, Wself_b, Wout_w, Wout_b, Wo_w, Wo_b, ss_edge_index, os_edge_index, fwd_edge_index, bwd_edge_index):
    raise NotImplementedError("write your pallas kernel here")


# speedup vs baseline: 3.0305x; 3.0305x over previous
import jax
import jax.numpy as jnp
from jax import lax
from jax.experimental import pallas as pl
from jax.experimental.pallas import tpu as pltpu
from jax.experimental.pallas import tpu_sc as plsc

N = 10000
E = 160000
D = 256
H = 128
NC = 2
NS = 16
EPT = E // NS
CH = 128
NCHUNK = -(-EPT // CH)
E_PAD = E + CH
EW = E // (NC * NS)
EW_PAD = EW + 16
N_PAD = 10240
QW = 16


_BLKN = 1000


def _prep_nodes_body(s_ref, o_ref, wss1_ref, wos1_ref,
                     tabss_ref, tabos_ref, tabo_ref):
    s_blk = s_ref[...]
    o_blk = o_ref[...]
    hss = jnp.dot(s_blk, wss1_ref[...], preferred_element_type=jnp.float32)
    hos = jnp.dot(o_blk, wos1_ref[...], preferred_element_type=jnp.float32)
    tabss_ref[...] = jnp.stack([hss[:, :H], hss[:, H:]])
    tabos_ref[...] = jnp.stack([hos[:, :H], hos[:, H:]])
    tabo_ref[...] = jnp.stack([o_blk[:, :H], o_blk[:, H:]])


def _prep_nodes(s_feat, o_feat, wss1, wos1):
    nb = N // _BLKN
    return pl.pallas_call(
        _prep_nodes_body,
        grid=(nb,),
        in_specs=[
            pl.BlockSpec((_BLKN, D), lambda i: (i, 0)),
            pl.BlockSpec((_BLKN, D), lambda i: (i, 0)),
            pl.BlockSpec((D, D), lambda i: (0, 0)),
            pl.BlockSpec((D, D), lambda i: (0, 0)),
        ],
        out_specs=[
            pl.BlockSpec((NC, _BLKN, H), lambda i: (0, i, 0)),
            pl.BlockSpec((NC, _BLKN, H), lambda i: (0, i, 0)),
            pl.BlockSpec((NC, _BLKN, H), lambda i: (0, i, 0)),
        ],
        out_shape=[
            jax.ShapeDtypeStruct((NC, N, H), jnp.float32),
            jax.ShapeDtypeStruct((NC, N, H), jnp.float32),
            jax.ShapeDtypeStruct((NC, N, H), jnp.float32),
        ],
    )(s_feat, o_feat, wss1, wos1)


def _prep_scalars_body(tabss_ref, tabos_ref, s_ref, wsw_ref, wsb_ref,
                       a1_ref, a2_ref, uss_ref, uos_ref, v_ref):
    a1_lo = a1_ref[pl.ds(0, H)][None, :]
    a1_hi = a1_ref[pl.ds(H, H)][None, :]
    uss_ref[...] = (jnp.sum(tabss_ref[0] * a1_lo, axis=1)
                    + jnp.sum(tabss_ref[1] * a1_hi, axis=1))
    uos_ref[...] = (jnp.sum(tabos_ref[0] * a1_lo, axis=1)
                    + jnp.sum(tabos_ref[1] * a1_hi, axis=1))
    a2 = a2_ref[...]
    m = jnp.sum(wsw_ref[...] * a2[None, :], axis=1)
    v_ref[...] = (jnp.sum(s_ref[...] * m[None, :], axis=1)
                  + jnp.sum(wsb_ref[...] * a2))


def _prep_scalars(tab_ss, tab_os, s_feat, ws_w, ws_b, a1, a2):
    return pl.pallas_call(
        _prep_scalars_body,
        out_shape=[
            jax.ShapeDtypeStruct((N,), jnp.float32),
            jax.ShapeDtypeStruct((N,), jnp.float32),
            jax.ShapeDtypeStruct((N,), jnp.float32),
        ],
    )(tab_ss, tab_os, s_feat, ws_w, ws_b, a1, a2)


_BLKE = 16000


def _prep_edges_body(ssf_ref, osf_ref, wss2_ref, wos2_ref, wssb_ref, wosb_ref,
                     a1_ref, ab_ref, wss_ref, wos_ref):
    a1 = a1_ref[...]
    g_ss = jnp.sum(wss2_ref[...] * a1[None, :], axis=1)
    g_os = jnp.sum(wos2_ref[...] * a1[None, :], axis=1)
    c_ss = jnp.sum(wssb_ref[...] * a1) + ab_ref[0]
    c_os = jnp.sum(wosb_ref[...] * a1) + ab_ref[0]
    sl = pl.ds(pl.program_id(0) * _BLKE, _BLKE)
    wss_ref[sl] = jnp.sum(ssf_ref[...] * g_ss[None, :], axis=1) + c_ss
    wos_ref[sl] = jnp.sum(osf_ref[...] * g_os[None, :], axis=1) + c_os


def _prep_edges(ss_feat, os_feat, wss2, wos2, wss_b, wos_b, a1, attn_b):
    nb = E // _BLKE
    return pl.pallas_call(
        _prep_edges_body,
        grid=(nb,),
        in_specs=[
            pl.BlockSpec((_BLKE, 10), lambda i: (i, 0)),
            pl.BlockSpec((_BLKE, 2), lambda i: (i, 0)),
            pl.BlockSpec((10, D), lambda i: (0, 0)),
            pl.BlockSpec((2, D), lambda i: (0, 0)),
            pl.BlockSpec((D,), lambda i: (0,)),
            pl.BlockSpec((D,), lambda i: (0,)),
            pl.BlockSpec((D,), lambda i: (0,)),
            pl.BlockSpec((1,), lambda i: (0,)),
        ],
        out_specs=[
            pl.BlockSpec((E,), lambda i: (0,)),
            pl.BlockSpec((E,), lambda i: (0,)),
        ],
        out_shape=[
            jax.ShapeDtypeStruct((E,), jnp.float32),
            jax.ShapeDtypeStruct((E,), jnp.float32),
        ],
    )(ss_feat, os_feat, wss2, wos2, wss_b, wos_b, a1, attn_b)


def _sc_t_body(uss_ref, uos_ref, v_ref, wss_ref, wos_ref,
               ss_src_ref, ss_dst_ref, os_src_ref, os_dst_ref,
               tss_out, tos_out,
               u1_v, u2_v, v_v, src_v, dst_v, w_v, t_v):
    c = lax.axis_index("c")
    s = lax.axis_index("s")
    wid = s * NC + c
    ebase = wid * EW

    pltpu.sync_copy(uss_ref, u1_v)
    pltpu.sync_copy(uos_ref, u2_v)
    pltpu.sync_copy(v_ref, v_v)

    def one(u_v, w_hbm, src_hbm, dst_hbm, t_out):
        src_v[pl.ds(EW_PAD - 16, 16)] = jnp.zeros((16,), jnp.int32)
        dst_v[pl.ds(EW_PAD - 16, 16)] = jnp.zeros((16,), jnp.int32)
        w_v[pl.ds(EW_PAD - 16, 16)] = jnp.zeros((16,), jnp.float32)
        pltpu.sync_copy(src_hbm.at[pl.ds(ebase, EW)], src_v.at[pl.ds(0, EW)])
        pltpu.sync_copy(dst_hbm.at[pl.ds(ebase, EW)], dst_v.at[pl.ds(0, EW)])
        pltpu.sync_copy(w_hbm.at[pl.ds(ebase, EW)], w_v.at[pl.ds(0, EW)])

        def grp(g, _):
            off = g * 16
            s16 = src_v[pl.ds(off, 16)]
            d16 = dst_v[pl.ds(off, 16)]
            u16 = plsc.load_gather(u_v, [s16])
            v16 = plsc.load_gather(v_v, [d16])
            x = u16 + v16 + w_v[pl.ds(off, 16)]
            e16 = jnp.where(x >= 0.0, x, x * jnp.float32(0.01))
            t_v[pl.ds(off, 16)] = jnp.exp(e16)
            return 0
        lax.fori_loop(0, EW_PAD // 16, grp, 0)
        pltpu.sync_copy(t_v.at[pl.ds(0, EW)], t_out.at[pl.ds(ebase, EW)])

    one(u1_v, wss_ref, ss_src_ref, ss_dst_ref, tss_out)
    one(u2_v, wos_ref, os_src_ref, os_dst_ref, tos_out)


def _sc_t(u_ss, u_os, v_s, w_ss, w_os, ss_src, ss_dst, os_src, os_dst):
    mesh = plsc.VectorSubcoreMesh(core_axis_name="c", subcore_axis_name="s")
    f32 = jnp.float32
    run = pl.kernel(
        _sc_t_body,
        compiler_params=pltpu.CompilerParams(needs_layout_passes=False),
        out_type=[
            jax.ShapeDtypeStruct((E,), f32),
            jax.ShapeDtypeStruct((E,), f32),
        ],
        mesh=mesh,
        scratch_types=[
            pltpu.VMEM((N,), f32),
            pltpu.VMEM((N,), f32),
            pltpu.VMEM((N,), f32),
            pltpu.VMEM((EW_PAD,), jnp.int32),
            pltpu.VMEM((EW_PAD,), jnp.int32),
            pltpu.VMEM((EW_PAD,), f32),
            pltpu.VMEM((EW_PAD,), f32),
        ],
    )
    return run(u_ss, u_os, v_s, w_ss, w_os, ss_src, ss_dst, os_src, os_dst)


def _sc_body(tabss_ref, tabos_ref, tabo_ref, tss_ref, tos_ref,
             qssf_ref, qosf_ref,
             ss_src_ref, ss_dst_ref, os_src_ref, os_dst_ref,
             f_src_ref, f_dst_ref, b_src_ref, b_dst_ref, zp_ref,
             pss_out, qss_out, pos_out, qos_out, af_out, ab_out,
             p_sh, src_c, dst_c, t_c, gidx_v, sidx_v, rows_v, qf_v, sem):
    c = lax.axis_index("c")
    s = lax.axis_index("s")
    c_n = c * N
    ebase = s * EPT
    NT = N_PAD // NS
    ntail = EPT - (NCHUNK - 1) * CH

    def zero_p():
        rb0 = s * NT
        pltpu.sync_copy(zp_ref.at[pl.ds(rb0, NT)], p_sh.at[pl.ds(rb0, NT)])

    def stage_chunk(i, src_hbm, dst_hbm, t_hbm):
        goff = ebase + i * CH
        pltpu.sync_copy(src_hbm.at[pl.ds(goff, CH)], src_c)
        pltpu.sync_copy(dst_hbm.at[pl.ds(goff, CH)], dst_c)
        if t_hbm is not None:
            pltpu.sync_copy(t_hbm.at[pl.ds(goff, CH)], t_c)

        def grp(g, _):
            off = g * 16
            gidx_v[pl.ds(off, 16)] = src_c[pl.ds(off, 16)] + c_n
            sidx_v[pl.ds(off, 16)] = dst_c[pl.ds(off, 16)]
            return 0
        lax.fori_loop(0, CH // 16, grp, 0)

        @pl.when(i == NCHUNK - 1)
        def _():
            def sent(g, _):
                sidx_v[pl.ds(ntail + g * 16, 16)] = \
                    jnp.full((16,), N, jnp.int32)
                return 0
            lax.fori_loop(0, (CH - ntail) // 16, sent, 0)

    def p_pass(tab_hbm, t_hbm, src_hbm, dst_hbm, p_out):
        zero_p()
        plsc.subcore_barrier()

        def chunk(i, _):
            stage_chunk(i, src_hbm, dst_hbm, t_hbm)
            pltpu.async_copy(tab_hbm.at[gidx_v], rows_v, sem).wait()
            if t_hbm is not None:
                def rowscale(j, _):
                    tb = plsc.load_gather(t_c, [jnp.full((16,), j, jnp.int32)])
                    for k in range(H // 16):
                        rows_v[j, pl.ds(k * 16, 16)] = \
                            rows_v[j, pl.ds(k * 16, 16)] * tb
                    return 0
                lax.fori_loop(0, CH, rowscale, 0)
            pltpu.sync_copy(rows_v, p_sh.at[sidx_v], add=True)
            return 0
        lax.fori_loop(0, NCHUNK, chunk, 0)
        plsc.subcore_barrier()
        pltpu.sync_copy(p_sh.at[pl.ds(s * NT, NT)],
                        p_out.at[c, pl.ds(s * NT, NT)])
        plsc.subcore_barrier()

    def q_chunks(qflat_hbm, t_hbm, src_hbm, dst_hbm):
        def chunk(i, _):
            stage_chunk(i, src_hbm, dst_hbm, t_hbm)
            goff = ebase + i * CH
            pltpu.sync_copy(qflat_hbm.at[pl.ds(goff * QW, CH * QW)], qf_v)

            def qscale(j, _):
                tb = plsc.load_gather(t_c, [jnp.full((16,), j, jnp.int32)])
                rows_v[j, pl.ds(0, 16)] = qf_v[pl.ds(j * QW, 16)] * tb
                return 0
            lax.fori_loop(0, CH, qscale, 0)
            pltpu.sync_copy(rows_v, p_sh.at[sidx_v], add=True)
            return 0
        lax.fori_loop(0, NCHUNK, chunk, 0)

    def q_pass():
        def zr(j, _):
            for k in range(H // 16):
                rows_v[j, pl.ds(k * 16, 16)] = jnp.zeros((16,), jnp.float32)
            return 0
        lax.fori_loop(0, CH, zr, 0)
        zero_p()
        plsc.subcore_barrier()

        @pl.when(c == 0)
        def _():
            q_chunks(qssf_ref, tss_ref, ss_src_ref, ss_dst_ref)

        @pl.when(c == 1)
        def _():
            q_chunks(qosf_ref, tos_ref, os_src_ref, os_dst_ref)
        plsc.subcore_barrier()

        @pl.when(c == 0)
        def _():
            pltpu.sync_copy(p_sh.at[pl.ds(s * NT, NT)],
                            qss_out.at[pl.ds(s * NT, NT)])

        @pl.when(c == 1)
        def _():
            pltpu.sync_copy(p_sh.at[pl.ds(s * NT, NT)],
                            qos_out.at[pl.ds(s * NT, NT)])
        plsc.subcore_barrier()

    p_pass(tabss_ref, tss_ref, ss_src_ref, ss_dst_ref, pss_out)
    p_pass(tabos_ref, tos_ref, os_src_ref, os_dst_ref, pos_out)
    q_pass()
    p_pass(tabo_ref, None, f_src_ref, f_dst_ref, af_out)
    p_pass(tabo_ref, None, b_src_ref, b_dst_ref, ab_out)


def _sc_agg(tab_ss, tab_os, tab_o, t_ss, t_os, qtab_ss, qtab_os,
            ss_src, ss_dst, os_src, os_dst, f_src, f_dst, b_src, b_dst):
    mesh = plsc.VectorSubcoreMesh(core_axis_name="c", subcore_axis_name="s")
    f32 = jnp.float32
    run = pl.kernel(
        _sc_body,
        compiler_params=pltpu.CompilerParams(needs_layout_passes=False),
        out_type=[
            jax.ShapeDtypeStruct((NC, N_PAD, H), f32),
            jax.ShapeDtypeStruct((N_PAD, H), f32),
            jax.ShapeDtypeStruct((NC, N_PAD, H), f32),
            jax.ShapeDtypeStruct((N_PAD, H), f32),
            jax.ShapeDtypeStruct((NC, N_PAD, H), f32),
            jax.ShapeDtypeStruct((NC, N_PAD, H), f32),
        ],
        mesh=mesh,
        scratch_types=[
            pltpu.VMEM_SHARED((N_PAD, H), f32),
            pltpu.VMEM((CH,), jnp.int32),
            pltpu.VMEM((CH,), jnp.int32),
            pltpu.VMEM((CH,), f32),
            pltpu.VMEM((CH,), jnp.int32),
            pltpu.VMEM((CH,), jnp.int32),
            pltpu.VMEM((CH, H), f32),
            pltpu.VMEM((CH * QW,), f32),
            pltpu.SemaphoreType.DMA,
        ],
    )
    zp = jnp.zeros((N_PAD, H), jnp.float32)
    return run(tab_ss.reshape(NC * N, H), tab_os.reshape(NC * N, H),
               tab_o.reshape(NC * N, H), t_ss, t_os,
               qtab_ss.reshape(-1), qtab_os.reshape(-1),
               ss_src, ss_dst, os_src, os_dst, f_src, f_dst, b_src, b_dst,
               zp)


_BLKO = 1000


def _post_body(pss_ref, qss_ref, pos_ref, qos_ref, af_ref, ab_ref, o_ref,
               wss2_ref, wssb_ref, wos2_ref, wosb_ref,
               win_w_ref, win_b_ref, wself_w_ref, wself_b_ref,
               wout_w_ref, wout_b_ref, wo_w_ref, wo_b_ref,
               z_ref, x_ref):
    def attn_half(p_ref, q_ref, w2_ref, b_ref, nf):
        p = jnp.concatenate([p_ref[0], p_ref[1]], axis=1)
        q = q_ref[...]
        den = q[:, nf:nf + 1]
        num = p + jnp.dot(q[:, :nf], w2_ref[...],
                          preferred_element_type=jnp.float32)
        return jnp.where(den > 0.0, num / den + b_ref[...][None, :], 0.0)

    z_ref[...] = (attn_half(pss_ref, qss_ref, wss2_ref, wssb_ref, 10)
                  + attn_half(pos_ref, qos_ref, wos2_ref, wosb_ref, 2))

    af = jnp.concatenate([af_ref[0], af_ref[1]], axis=1)
    ab = jnp.concatenate([ab_ref[0], ab_ref[1]], axis=1)
    o_blk = o_ref[...]
    h_in = jax.nn.relu(jnp.dot(af, win_w_ref[...],
                               preferred_element_type=jnp.float32)
                       + win_b_ref[...][None, :])
    h_self = jax.nn.relu(jnp.dot(o_blk, wself_w_ref[...],
                                 preferred_element_type=jnp.float32)
                         + wself_b_ref[...][None, :])
    h_out = jax.nn.relu(jnp.dot(ab, wout_w_ref[...],
                                preferred_element_type=jnp.float32)
                        + wout_b_ref[...][None, :])
    wo = wo_w_ref[...]
    x_ref[...] = (jnp.dot(h_in, wo[:D], preferred_element_type=jnp.float32)
                  + jnp.dot(h_self, wo[D:2 * D],
                            preferred_element_type=jnp.float32)
                  + jnp.dot(h_out, wo[2 * D:],
                            preferred_element_type=jnp.float32)
                  + wo_b_ref[...][None, :])


def _post(p_ss, q_ss, p_os, q_os, a_f, a_b, o_feat,
          wss2, wss_b, wos2, wos_b, win_w, win_b, wself_w, wself_b,
          wout_w, wout_b, wo_w, wo_b):
    nb = N // _BLKO

    def vec(d):
        return pl.BlockSpec((d,), lambda i: (0,))

    return pl.pallas_call(
        _post_body,
        grid=(nb,),
        in_specs=[
            pl.BlockSpec((NC, _BLKO, H), lambda i: (0, i, 0)),
            pl.BlockSpec((_BLKO, H), lambda i: (i, 0)),
            pl.BlockSpec((NC, _BLKO, H), lambda i: (0, i, 0)),
            pl.BlockSpec((_BLKO, H), lambda i: (i, 0)),
            pl.BlockSpec((NC, _BLKO, H), lambda i: (0, i, 0)),
            pl.BlockSpec((NC, _BLKO, H), lambda i: (0, i, 0)),
            pl.BlockSpec((_BLKO, D), lambda i: (i, 0)),
            pl.BlockSpec((10, D), lambda i: (0, 0)),
            vec(D),
            pl.BlockSpec((2, D), lambda i: (0, 0)),
            vec(D),
            pl.BlockSpec((D, D), lambda i: (0, 0)),
            vec(D),
            pl.BlockSpec((D, D), lambda i: (0, 0)),
            vec(D),
            pl.BlockSpec((D, D), lambda i: (0, 0)),
            vec(D),
            pl.BlockSpec((3 * D, D), lambda i: (0, 0)),
            vec(D),
        ],
        out_specs=[
            pl.BlockSpec((_BLKO, D), lambda i: (i, 0)),
            pl.BlockSpec((_BLKO, D), lambda i: (i, 0)),
        ],
        out_shape=[
            jax.ShapeDtypeStruct((N, D), jnp.float32),
            jax.ShapeDtypeStruct((N, D), jnp.float32),
        ],
    )(p_ss, q_ss, p_os, q_os, a_f, a_b, o_feat,
      wss2, wss_b, wos2, wos_b, win_w, win_b, wself_w, wself_b,
      wout_w, wout_b, wo_w, wo_b)


def kernel(s_feat, o_feat, ss_feat, os_feat, Ws_w, Ws_b, Wos_w, Wos_b,
           Wss_w, Wss_b, attn_w, attn_b, Win_w, Win_b, Wself_w, Wself_b,
           Wout_w, Wout_b, Wo_w, Wo_b,
           ss_edge_index, os_edge_index, fwd_edge_index, bwd_edge_index):
    a1 = attn_w[:D, 0]
    a2 = attn_w[D:, 0]

    tab_ss, tab_os, tab_o = _prep_nodes(s_feat, o_feat, Wss_w[:D], Wos_w[:D])
    u_ss, u_os, v_s = _prep_scalars(tab_ss, tab_os, s_feat, Ws_w, Ws_b,
                                    a1, a2)

    w_ss, w_os = _prep_edges(
        ss_feat, os_feat, Wss_w[D:], Wos_w[D:], Wss_b, Wos_b, a1, attn_b)

    ones = jnp.ones((E, 1), jnp.float32)
    pad_e = E_PAD - E
    qtab_ss = jnp.pad(
        jnp.concatenate([ss_feat, ones, jnp.zeros((E, QW - 11), jnp.float32)],
                        axis=1), ((0, pad_e), (0, 0)))
    qtab_os = jnp.pad(
        jnp.concatenate([os_feat, ones, jnp.zeros((E, QW - 3), jnp.float32)],
                        axis=1), ((0, pad_e), (0, 0)))

    idx = [jnp.pad(a.astype(jnp.int32), (0, CH)) for a in
           (ss_edge_index[0], ss_edge_index[1], os_edge_index[0],
            os_edge_index[1], fwd_edge_index[0], fwd_edge_index[1],
            bwd_edge_index[0], bwd_edge_index[1])]

    t_ss, t_os = _sc_t(u_ss, u_os, v_s, w_ss, w_os,
                       idx[0], idx[1], idx[2], idx[3])
    t_ss = jnp.pad(t_ss, (0, CH))
    t_os = jnp.pad(t_os, (0, CH))

    p_ss, q_ss, p_os, q_os, a_f, a_b = _sc_agg(
        tab_ss, tab_os, tab_o, t_ss, t_os, qtab_ss, qtab_os, *idx)

    z, x = _post(p_ss[:, :N], q_ss[:N], p_os[:, :N], q_os[:N],
                 a_f[:, :N], a_b[:, :N], o_feat,
                 Wss_w[D:], Wss_b, Wos_w[D:], Wos_b, Win_w, Win_b,
                 Wself_w, Wself_b, Wout_w, Wout_b, Wo_w, Wo_b)
    return z, x


# speedup vs baseline: 3.0749x; 1.0146x over previous
import jax
import jax.numpy as jnp
from jax import lax
from jax.experimental import pallas as pl
from jax.experimental.pallas import tpu as pltpu
from jax.experimental.pallas import tpu_sc as plsc

N = 10000
E = 160000
D = 256
H = 128
NC = 2
NS = 16
EPT = E // NS
CH = 128
NCHUNK = -(-EPT // CH)
E_PAD = E + CH
EW = E // (NC * NS)
EW_PAD = EW + 16
N_PAD = 10240
QW = 16


_BLKN = 1000


def _prep_nodes_body(s_ref, o_ref, wss1_ref, wos1_ref,
                     tabss_ref, tabos_ref, tabo_ref):
    s_blk = s_ref[...]
    o_blk = o_ref[...]
    hss = jnp.dot(s_blk, wss1_ref[...], preferred_element_type=jnp.float32)
    hos = jnp.dot(o_blk, wos1_ref[...], preferred_element_type=jnp.float32)
    tabss_ref[...] = jnp.stack([hss[:, :H], hss[:, H:]])
    tabos_ref[...] = jnp.stack([hos[:, :H], hos[:, H:]])
    tabo_ref[...] = jnp.stack([o_blk[:, :H], o_blk[:, H:]])


def _prep_nodes(s_feat, o_feat, wss1, wos1):
    nb = N // _BLKN
    return pl.pallas_call(
        _prep_nodes_body,
        grid=(nb,),
        in_specs=[
            pl.BlockSpec((_BLKN, D), lambda i: (i, 0)),
            pl.BlockSpec((_BLKN, D), lambda i: (i, 0)),
            pl.BlockSpec((D, D), lambda i: (0, 0)),
            pl.BlockSpec((D, D), lambda i: (0, 0)),
        ],
        out_specs=[
            pl.BlockSpec((NC, _BLKN, H), lambda i: (0, i, 0)),
            pl.BlockSpec((NC, _BLKN, H), lambda i: (0, i, 0)),
            pl.BlockSpec((NC, _BLKN, H), lambda i: (0, i, 0)),
        ],
        out_shape=[
            jax.ShapeDtypeStruct((NC, N, H), jnp.float32),
            jax.ShapeDtypeStruct((NC, N, H), jnp.float32),
            jax.ShapeDtypeStruct((NC, N, H), jnp.float32),
        ],
    )(s_feat, o_feat, wss1, wos1)


def _prep_scalars_body(tabss_ref, tabos_ref, s_ref, wsw_ref, wsb_ref,
                       a1_ref, a2_ref, uss_ref, uos_ref, v_ref):
    a1_lo = a1_ref[pl.ds(0, H)][None, :]
    a1_hi = a1_ref[pl.ds(H, H)][None, :]
    uss_ref[...] = (jnp.sum(tabss_ref[0] * a1_lo, axis=1)
                    + jnp.sum(tabss_ref[1] * a1_hi, axis=1))
    uos_ref[...] = (jnp.sum(tabos_ref[0] * a1_lo, axis=1)
                    + jnp.sum(tabos_ref[1] * a1_hi, axis=1))
    a2 = a2_ref[...]
    m = jnp.sum(wsw_ref[...] * a2[None, :], axis=1)
    v_ref[...] = (jnp.sum(s_ref[...] * m[None, :], axis=1)
                  + jnp.sum(wsb_ref[...] * a2))


def _prep_scalars(tab_ss, tab_os, s_feat, ws_w, ws_b, a1, a2):
    return pl.pallas_call(
        _prep_scalars_body,
        out_shape=[
            jax.ShapeDtypeStruct((N,), jnp.float32),
            jax.ShapeDtypeStruct((N,), jnp.float32),
            jax.ShapeDtypeStruct((N,), jnp.float32),
        ],
    )(tab_ss, tab_os, s_feat, ws_w, ws_b, a1, a2)


_BLKE = 16000


def _prep_edges_body(ssf_ref, osf_ref, wss2_ref, wos2_ref, wssb_ref, wosb_ref,
                     a1_ref, ab_ref, wss_ref, wos_ref):
    a1 = a1_ref[...]
    g_ss = jnp.sum(wss2_ref[...] * a1[None, :], axis=1)
    g_os = jnp.sum(wos2_ref[...] * a1[None, :], axis=1)
    c_ss = jnp.sum(wssb_ref[...] * a1) + ab_ref[0]
    c_os = jnp.sum(wosb_ref[...] * a1) + ab_ref[0]
    sl = pl.ds(pl.program_id(0) * _BLKE, _BLKE)
    wss_ref[sl] = jnp.sum(ssf_ref[...] * g_ss[None, :], axis=1) + c_ss
    wos_ref[sl] = jnp.sum(osf_ref[...] * g_os[None, :], axis=1) + c_os


def _prep_edges(ss_feat, os_feat, wss2, wos2, wss_b, wos_b, a1, attn_b):
    nb = E // _BLKE
    return pl.pallas_call(
        _prep_edges_body,
        grid=(nb,),
        in_specs=[
            pl.BlockSpec((_BLKE, 10), lambda i: (i, 0)),
            pl.BlockSpec((_BLKE, 2), lambda i: (i, 0)),
            pl.BlockSpec((10, D), lambda i: (0, 0)),
            pl.BlockSpec((2, D), lambda i: (0, 0)),
            pl.BlockSpec((D,), lambda i: (0,)),
            pl.BlockSpec((D,), lambda i: (0,)),
            pl.BlockSpec((D,), lambda i: (0,)),
            pl.BlockSpec((1,), lambda i: (0,)),
        ],
        out_specs=[
            pl.BlockSpec((E,), lambda i: (0,)),
            pl.BlockSpec((E,), lambda i: (0,)),
        ],
        out_shape=[
            jax.ShapeDtypeStruct((E,), jnp.float32),
            jax.ShapeDtypeStruct((E,), jnp.float32),
        ],
    )(ss_feat, os_feat, wss2, wos2, wss_b, wos_b, a1, attn_b)


def _sc_t_body(uss_ref, uos_ref, v_ref, wss_ref, wos_ref,
               ss_src_ref, ss_dst_ref, os_src_ref, os_dst_ref,
               tss_out, tos_out,
               u1_v, u2_v, v_v, src_v, dst_v, w_v, t_v):
    c = lax.axis_index("c")
    s = lax.axis_index("s")
    wid = s * NC + c
    ebase = wid * EW

    pltpu.sync_copy(uss_ref, u1_v)
    pltpu.sync_copy(uos_ref, u2_v)
    pltpu.sync_copy(v_ref, v_v)

    def one(u_v, w_hbm, src_hbm, dst_hbm, t_out):
        src_v[pl.ds(EW_PAD - 16, 16)] = jnp.zeros((16,), jnp.int32)
        dst_v[pl.ds(EW_PAD - 16, 16)] = jnp.zeros((16,), jnp.int32)
        w_v[pl.ds(EW_PAD - 16, 16)] = jnp.zeros((16,), jnp.float32)
        pltpu.sync_copy(src_hbm.at[pl.ds(ebase, EW)], src_v.at[pl.ds(0, EW)])
        pltpu.sync_copy(dst_hbm.at[pl.ds(ebase, EW)], dst_v.at[pl.ds(0, EW)])
        pltpu.sync_copy(w_hbm.at[pl.ds(ebase, EW)], w_v.at[pl.ds(0, EW)])

        def grp(g, _):
            off = g * 16
            s16 = src_v[pl.ds(off, 16)]
            d16 = dst_v[pl.ds(off, 16)]
            u16 = plsc.load_gather(u_v, [s16])
            v16 = plsc.load_gather(v_v, [d16])
            x = u16 + v16 + w_v[pl.ds(off, 16)]
            e16 = jnp.where(x >= 0.0, x, x * jnp.float32(0.01))
            t_v[pl.ds(off, 16)] = jnp.exp(e16)
            return 0
        lax.fori_loop(0, EW_PAD // 16, grp, 0)
        pltpu.sync_copy(t_v.at[pl.ds(0, EW)], t_out.at[pl.ds(ebase, EW)])

    one(u1_v, wss_ref, ss_src_ref, ss_dst_ref, tss_out)
    one(u2_v, wos_ref, os_src_ref, os_dst_ref, tos_out)


def _sc_t(u_ss, u_os, v_s, w_ss, w_os, ss_src, ss_dst, os_src, os_dst):
    mesh = plsc.VectorSubcoreMesh(core_axis_name="c", subcore_axis_name="s")
    f32 = jnp.float32
    run = pl.kernel(
        _sc_t_body,
        compiler_params=pltpu.CompilerParams(needs_layout_passes=False),
        out_type=[
            jax.ShapeDtypeStruct((E,), f32),
            jax.ShapeDtypeStruct((E,), f32),
        ],
        mesh=mesh,
        scratch_types=[
            pltpu.VMEM((N,), f32),
            pltpu.VMEM((N,), f32),
            pltpu.VMEM((N,), f32),
            pltpu.VMEM((EW_PAD,), jnp.int32),
            pltpu.VMEM((EW_PAD,), jnp.int32),
            pltpu.VMEM((EW_PAD,), f32),
            pltpu.VMEM((EW_PAD,), f32),
        ],
    )
    return run(u_ss, u_os, v_s, w_ss, w_os, ss_src, ss_dst, os_src, os_dst)


def _sc_body(tabss_ref, tabos_ref, tabo_ref, tss_ref, tos_ref,
             qssf_ref, qosf_ref,
             ss_src_ref, ss_dst_ref, os_src_ref, os_dst_ref,
             f_src_ref, f_dst_ref, b_src_ref, b_dst_ref, zp_ref,
             pss_out, qss_out, pos_out, qos_out, af_out, ab_out,
             p_sh, src_c, dst_c, t_c, gidx_v, sidx_v, rows_v, qf_v, sem):
    c = lax.axis_index("c")
    s = lax.axis_index("s")
    c_n = c * N
    ebase = s * EPT
    NT = N_PAD // NS
    ntail = EPT - (NCHUNK - 1) * CH

    def zero_p():
        rb0 = s * NT
        pltpu.sync_copy(zp_ref.at[pl.ds(rb0, NT)], p_sh.at[pl.ds(rb0, NT)])

    def stage_chunk(i, src_hbm, dst_hbm, t_hbm):
        goff = ebase + i * CH
        pltpu.sync_copy(src_hbm.at[pl.ds(goff, CH)], src_c)
        pltpu.sync_copy(dst_hbm.at[pl.ds(goff, CH)], dst_c)
        if t_hbm is not None:
            pltpu.sync_copy(t_hbm.at[pl.ds(goff, CH)], t_c)

        def grp(g, _):
            off = g * 16
            gidx_v[pl.ds(off, 16)] = src_c[pl.ds(off, 16)] + c_n
            sidx_v[pl.ds(off, 16)] = dst_c[pl.ds(off, 16)]
            return 0
        lax.fori_loop(0, CH // 16, grp, 0)

        @pl.when(i == NCHUNK - 1)
        def _():
            def sent(g, _):
                sidx_v[pl.ds(ntail + g * 16, 16)] = \
                    jnp.full((16,), N, jnp.int32)
                return 0
            lax.fori_loop(0, (CH - ntail) // 16, sent, 0)

    def p_pass(tab_hbm, t_hbm, src_hbm, dst_hbm, p_out):
        zero_p()
        plsc.subcore_barrier()

        def chunk(i, _):
            stage_chunk(i, src_hbm, dst_hbm, t_hbm)
            pltpu.async_copy(tab_hbm.at[gidx_v], rows_v, sem).wait()
            if t_hbm is not None:
                def rowscale(j, _):
                    tb = plsc.load_gather(t_c, [jnp.full((16,), j, jnp.int32)])
                    for k in range(H // 16):
                        rows_v[j, pl.ds(k * 16, 16)] = \
                            rows_v[j, pl.ds(k * 16, 16)] * tb
                    return 0
                lax.fori_loop(0, CH, rowscale, 0)
            pltpu.sync_copy(rows_v, p_sh.at[sidx_v], add=True)
            return 0
        lax.fori_loop(0, NCHUNK, chunk, 0)
        plsc.subcore_barrier()
        pltpu.sync_copy(p_sh.at[pl.ds(s * NT, NT)],
                        p_out.at[c, pl.ds(s * NT, NT)])
        plsc.subcore_barrier()

    def q_chunks(qflat_hbm, t_hbm, src_hbm, dst_hbm):
        def chunk(i, _):
            stage_chunk(i, src_hbm, dst_hbm, t_hbm)
            goff = ebase + i * CH
            pltpu.sync_copy(qflat_hbm.at[pl.ds(goff * QW, CH * QW)], qf_v)

            def qscale(j, _):
                tb = plsc.load_gather(t_c, [jnp.full((16,), j, jnp.int32)])
                rows_v[j, pl.ds(0, 16)] = qf_v[pl.ds(j * QW, 16)] * tb
                return 0
            lax.fori_loop(0, CH, qscale, 0)
            pltpu.sync_copy(rows_v, p_sh.at[sidx_v], add=True)
            return 0
        lax.fori_loop(0, NCHUNK, chunk, 0)

    def q_pass():
        def zr(j, _):
            for k in range(H // 16):
                rows_v[j, pl.ds(k * 16, 16)] = jnp.zeros((16,), jnp.float32)
            return 0
        lax.fori_loop(0, CH, zr, 0)
        zero_p()
        plsc.subcore_barrier()

        @pl.when(c == 0)
        def _():
            q_chunks(qssf_ref, tss_ref, ss_src_ref, ss_dst_ref)

        @pl.when(c == 1)
        def _():
            q_chunks(qosf_ref, tos_ref, os_src_ref, os_dst_ref)
        plsc.subcore_barrier()

        @pl.when(c == 0)
        def _():
            pltpu.sync_copy(p_sh.at[pl.ds(s * NT, NT)],
                            qss_out.at[pl.ds(s * NT, NT)])

        @pl.when(c == 1)
        def _():
            pltpu.sync_copy(p_sh.at[pl.ds(s * NT, NT)],
                            qos_out.at[pl.ds(s * NT, NT)])
        plsc.subcore_barrier()

    p_pass(tabss_ref, tss_ref, ss_src_ref, ss_dst_ref, pss_out)
    p_pass(tabos_ref, tos_ref, os_src_ref, os_dst_ref, pos_out)
    q_pass()
    p_pass(tabo_ref, None, f_src_ref, f_dst_ref, af_out)
    p_pass(tabo_ref, None, b_src_ref, b_dst_ref, ab_out)


def _sc_agg(tab_ss, tab_os, tab_o, t_ss, t_os, qtab_ss, qtab_os,
            ss_src, ss_dst, os_src, os_dst, f_src, f_dst, b_src, b_dst):
    mesh = plsc.VectorSubcoreMesh(core_axis_name="c", subcore_axis_name="s")
    f32 = jnp.float32
    run = pl.kernel(
        _sc_body,
        compiler_params=pltpu.CompilerParams(needs_layout_passes=False),
        out_type=[
            jax.ShapeDtypeStruct((NC, N_PAD, H), f32),
            jax.ShapeDtypeStruct((N_PAD, H), f32),
            jax.ShapeDtypeStruct((NC, N_PAD, H), f32),
            jax.ShapeDtypeStruct((N_PAD, H), f32),
            jax.ShapeDtypeStruct((NC, N_PAD, H), f32),
            jax.ShapeDtypeStruct((NC, N_PAD, H), f32),
        ],
        mesh=mesh,
        scratch_types=[
            pltpu.VMEM_SHARED((N_PAD, H), f32),
            pltpu.VMEM((CH,), jnp.int32),
            pltpu.VMEM((CH,), jnp.int32),
            pltpu.VMEM((CH,), f32),
            pltpu.VMEM((CH,), jnp.int32),
            pltpu.VMEM((CH,), jnp.int32),
            pltpu.VMEM((CH, H), f32),
            pltpu.VMEM((CH * QW,), f32),
            pltpu.SemaphoreType.DMA,
        ],
    )
    zp = jnp.zeros((N_PAD, H), jnp.float32)
    return run(tab_ss.reshape(NC * N, H), tab_os.reshape(NC * N, H),
               tab_o.reshape(NC * N, H), t_ss, t_os,
               qtab_ss.reshape(-1), qtab_os.reshape(-1),
               ss_src, ss_dst, os_src, os_dst, f_src, f_dst, b_src, b_dst,
               zp)


_BLKO = 1000


def _post_body(pss_ref, qss_ref, pos_ref, qos_ref, af_ref, ab_ref, o_ref,
               wss2_ref, wssb_ref, wos2_ref, wosb_ref,
               win_w_ref, win_b_ref, wself_w_ref, wself_b_ref,
               wout_w_ref, wout_b_ref, wo_w_ref, wo_b_ref,
               z_ref, x_ref):
    def attn_half(p_ref, q_ref, w2_ref, b_ref, nf):
        p = jnp.concatenate([p_ref[0], p_ref[1]], axis=1)
        q = q_ref[...]
        den = q[:, nf:nf + 1]
        num = p + jnp.dot(q[:, :nf], w2_ref[...],
                          preferred_element_type=jnp.float32)
        return jnp.where(den > 0.0, num / den + b_ref[...][None, :], 0.0)

    z_ref[...] = (attn_half(pss_ref, qss_ref, wss2_ref, wssb_ref, 10)
                  + attn_half(pos_ref, qos_ref, wos2_ref, wosb_ref, 2))

    af = jnp.concatenate([af_ref[0], af_ref[1]], axis=1)
    ab = jnp.concatenate([ab_ref[0], ab_ref[1]], axis=1)
    o_blk = o_ref[...]
    h_in = jax.nn.relu(jnp.dot(af, win_w_ref[...],
                               preferred_element_type=jnp.float32)
                       + win_b_ref[...][None, :])
    h_self = jax.nn.relu(jnp.dot(o_blk, wself_w_ref[...],
                                 preferred_element_type=jnp.float32)
                         + wself_b_ref[...][None, :])
    h_out = jax.nn.relu(jnp.dot(ab, wout_w_ref[...],
                                preferred_element_type=jnp.float32)
                        + wout_b_ref[...][None, :])
    wo = wo_w_ref[...]
    x_ref[...] = (jnp.dot(h_in, wo[:D], preferred_element_type=jnp.float32)
                  + jnp.dot(h_self, wo[D:2 * D],
                            preferred_element_type=jnp.float32)
                  + jnp.dot(h_out, wo[2 * D:],
                            preferred_element_type=jnp.float32)
                  + wo_b_ref[...][None, :])


def _post(p_ss, q_ss, p_os, q_os, a_f, a_b, o_feat,
          wss2, wss_b, wos2, wos_b, win_w, win_b, wself_w, wself_b,
          wout_w, wout_b, wo_w, wo_b):
    nb = N // _BLKO

    def vec(d):
        return pl.BlockSpec((d,), lambda i: (0,))

    return pl.pallas_call(
        _post_body,
        grid=(nb,),
        in_specs=[
            pl.BlockSpec((NC, _BLKO, H), lambda i: (0, i, 0)),
            pl.BlockSpec((_BLKO, H), lambda i: (i, 0)),
            pl.BlockSpec((NC, _BLKO, H), lambda i: (0, i, 0)),
            pl.BlockSpec((_BLKO, H), lambda i: (i, 0)),
            pl.BlockSpec((NC, _BLKO, H), lambda i: (0, i, 0)),
            pl.BlockSpec((NC, _BLKO, H), lambda i: (0, i, 0)),
            pl.BlockSpec((_BLKO, D), lambda i: (i, 0)),
            pl.BlockSpec((10, D), lambda i: (0, 0)),
            vec(D),
            pl.BlockSpec((2, D), lambda i: (0, 0)),
            vec(D),
            pl.BlockSpec((D, D), lambda i: (0, 0)),
            vec(D),
            pl.BlockSpec((D, D), lambda i: (0, 0)),
            vec(D),
            pl.BlockSpec((D, D), lambda i: (0, 0)),
            vec(D),
            pl.BlockSpec((3 * D, D), lambda i: (0, 0)),
            vec(D),
        ],
        out_specs=[
            pl.BlockSpec((_BLKO, D), lambda i: (i, 0)),
            pl.BlockSpec((_BLKO, D), lambda i: (i, 0)),
        ],
        out_shape=[
            jax.ShapeDtypeStruct((N, D), jnp.float32),
            jax.ShapeDtypeStruct((N, D), jnp.float32),
        ],
    )(p_ss, q_ss, p_os, q_os, a_f, a_b, o_feat,
      wss2, wss_b, wos2, wos_b, win_w, win_b, wself_w, wself_b,
      wout_w, wout_b, wo_w, wo_b)


def kernel(s_feat, o_feat, ss_feat, os_feat, Ws_w, Ws_b, Wos_w, Wos_b,
           Wss_w, Wss_b, attn_w, attn_b, Win_w, Win_b, Wself_w, Wself_b,
           Wout_w, Wout_b, Wo_w, Wo_b,
           ss_edge_index, os_edge_index, fwd_edge_index, bwd_edge_index):
    a1 = attn_w[:D, 0]
    a2 = attn_w[D:, 0]

    tab_ss, tab_os, tab_o = _prep_nodes(s_feat, o_feat, Wss_w[:D], Wos_w[:D])
    u_ss, u_os, v_s = _prep_scalars(tab_ss, tab_os, s_feat, Ws_w, Ws_b,
                                    a1, a2)

    w_ss, w_os = _prep_edges(
        ss_feat, os_feat, Wss_w[D:], Wos_w[D:], Wss_b, Wos_b, a1, attn_b)

    ones = jnp.ones((E, 1), jnp.float32)
    pad_e = E_PAD - E
    qtab_ss = jnp.pad(
        jnp.concatenate([ss_feat, ones, jnp.zeros((E, QW - 11), jnp.float32)],
                        axis=1), ((0, pad_e), (0, 0)))
    qtab_os = jnp.pad(
        jnp.concatenate([os_feat, ones, jnp.zeros((E, QW - 3), jnp.float32)],
                        axis=1), ((0, pad_e), (0, 0)))

    idx = [jnp.pad(a.astype(jnp.int32), (0, CH)) for a in
           (ss_edge_index[0], ss_edge_index[1], os_edge_index[0],
            os_edge_index[1], fwd_edge_index[0], fwd_edge_index[1],
            bwd_edge_index[0], bwd_edge_index[1])]

    t_ss, t_os = _sc_t(u_ss, u_os, v_s, w_ss, w_os,
                       idx[0], idx[1], idx[2], idx[3])
    t_ss = jnp.pad(t_ss, (0, CH))
    t_os = jnp.pad(t_os, (0, CH))

    p_ss, q_ss, p_os, q_os, a_f, a_b = _sc_agg(
        tab_ss, tab_os, tab_o, t_ss, t_os, qtab_ss, qtab_os, *idx)

    z, x = _post(p_ss, q_ss, p_os, q_os, a_f, a_b, o_feat,
                 Wss_w[D:], Wss_b, Wos_w[D:], Wos_b, Win_w, Win_b,
                 Wself_w, Wself_b, Wout_w, Wout_b, Wo_w, Wo_b)
    return z, x


# speedup vs baseline: 3.7466x; 1.2185x over previous
import jax
import jax.numpy as jnp
from jax import lax
from jax.experimental import pallas as pl
from jax.experimental.pallas import tpu as pltpu
from jax.experimental.pallas import tpu_sc as plsc

N = 10000
E = 160000
D = 256
H = 128
NC = 2
NS = 16
EPT = E // NS
CH = 128
NCHUNK = -(-EPT // CH)
E_PAD = E + CH
EW = E // (NC * NS)
EW_PAD = EW + 16
N_PAD = 10240
QW = 16


_BLKN = 1000


def _prep_nodes_body(s_ref, o_ref, wss1_ref, wos1_ref,
                     tabss_ref, tabos_ref, tabo_ref):
    s_blk = s_ref[...]
    o_blk = o_ref[...]
    hss = jnp.dot(s_blk, wss1_ref[...], preferred_element_type=jnp.float32)
    hos = jnp.dot(o_blk, wos1_ref[...], preferred_element_type=jnp.float32)
    tabss_ref[...] = jnp.stack([hss[:, :H], hss[:, H:]])
    tabos_ref[...] = jnp.stack([hos[:, :H], hos[:, H:]])
    tabo_ref[...] = jnp.stack([o_blk[:, :H], o_blk[:, H:]])


def _prep_nodes(s_feat, o_feat, wss1, wos1):
    nb = N // _BLKN
    return pl.pallas_call(
        _prep_nodes_body,
        grid=(nb,),
        in_specs=[
            pl.BlockSpec((_BLKN, D), lambda i: (i, 0)),
            pl.BlockSpec((_BLKN, D), lambda i: (i, 0)),
            pl.BlockSpec((D, D), lambda i: (0, 0)),
            pl.BlockSpec((D, D), lambda i: (0, 0)),
        ],
        out_specs=[
            pl.BlockSpec((NC, _BLKN, H), lambda i: (0, i, 0)),
            pl.BlockSpec((NC, _BLKN, H), lambda i: (0, i, 0)),
            pl.BlockSpec((NC, _BLKN, H), lambda i: (0, i, 0)),
        ],
        out_shape=[
            jax.ShapeDtypeStruct((NC, N, H), jnp.float32),
            jax.ShapeDtypeStruct((NC, N, H), jnp.float32),
            jax.ShapeDtypeStruct((NC, N, H), jnp.float32),
        ],
    )(s_feat, o_feat, wss1, wos1)


def _prep_scalars_body(tabss_ref, tabos_ref, s_ref, wsw_ref, wsb_ref,
                       a1_ref, a2_ref, uss_ref, uos_ref, v_ref):
    a1_lo = a1_ref[pl.ds(0, H)][None, :]
    a1_hi = a1_ref[pl.ds(H, H)][None, :]
    uss_ref[...] = (jnp.sum(tabss_ref[0] * a1_lo, axis=1)
                    + jnp.sum(tabss_ref[1] * a1_hi, axis=1))
    uos_ref[...] = (jnp.sum(tabos_ref[0] * a1_lo, axis=1)
                    + jnp.sum(tabos_ref[1] * a1_hi, axis=1))
    a2 = a2_ref[...]
    m = jnp.sum(wsw_ref[...] * a2[None, :], axis=1)
    v_ref[...] = (jnp.sum(s_ref[...] * m[None, :], axis=1)
                  + jnp.sum(wsb_ref[...] * a2))


def _prep_scalars(tab_ss, tab_os, s_feat, ws_w, ws_b, a1, a2):
    return pl.pallas_call(
        _prep_scalars_body,
        out_shape=[
            jax.ShapeDtypeStruct((N,), jnp.float32),
            jax.ShapeDtypeStruct((N,), jnp.float32),
            jax.ShapeDtypeStruct((N,), jnp.float32),
        ],
    )(tab_ss, tab_os, s_feat, ws_w, ws_b, a1, a2)


_BLKE = 16000


def _prep_edges_body(ssf_ref, osf_ref, wss2_ref, wos2_ref, wssb_ref, wosb_ref,
                     a1_ref, ab_ref, wss_ref, wos_ref):
    a1 = a1_ref[...]
    g_ss = jnp.sum(wss2_ref[...] * a1[None, :], axis=1)
    g_os = jnp.sum(wos2_ref[...] * a1[None, :], axis=1)
    c_ss = jnp.sum(wssb_ref[...] * a1) + ab_ref[0]
    c_os = jnp.sum(wosb_ref[...] * a1) + ab_ref[0]
    sl = pl.ds(pl.program_id(0) * _BLKE, _BLKE)
    wss_ref[sl] = jnp.sum(ssf_ref[...] * g_ss[None, :], axis=1) + c_ss
    wos_ref[sl] = jnp.sum(osf_ref[...] * g_os[None, :], axis=1) + c_os


def _prep_edges(ss_feat, os_feat, wss2, wos2, wss_b, wos_b, a1, attn_b):
    nb = E // _BLKE
    return pl.pallas_call(
        _prep_edges_body,
        grid=(nb,),
        in_specs=[
            pl.BlockSpec((_BLKE, 10), lambda i: (i, 0)),
            pl.BlockSpec((_BLKE, 2), lambda i: (i, 0)),
            pl.BlockSpec((10, D), lambda i: (0, 0)),
            pl.BlockSpec((2, D), lambda i: (0, 0)),
            pl.BlockSpec((D,), lambda i: (0,)),
            pl.BlockSpec((D,), lambda i: (0,)),
            pl.BlockSpec((D,), lambda i: (0,)),
            pl.BlockSpec((1,), lambda i: (0,)),
        ],
        out_specs=[
            pl.BlockSpec((E,), lambda i: (0,)),
            pl.BlockSpec((E,), lambda i: (0,)),
        ],
        out_shape=[
            jax.ShapeDtypeStruct((E,), jnp.float32),
            jax.ShapeDtypeStruct((E,), jnp.float32),
        ],
    )(ss_feat, os_feat, wss2, wos2, wss_b, wos_b, a1, attn_b)


def _sc_t_body(uss_ref, uos_ref, v_ref, wss_ref, wos_ref,
               ss_src_ref, ss_dst_ref, os_src_ref, os_dst_ref,
               tss_out, tos_out,
               u1_v, u2_v, v_v, src_v, dst_v, w_v, t_v):
    c = lax.axis_index("c")
    s = lax.axis_index("s")
    wid = s * NC + c
    ebase = wid * EW

    pltpu.sync_copy(uss_ref, u1_v)
    pltpu.sync_copy(uos_ref, u2_v)
    pltpu.sync_copy(v_ref, v_v)

    def one(u_v, w_hbm, src_hbm, dst_hbm, t_out):
        src_v[pl.ds(EW_PAD - 16, 16)] = jnp.zeros((16,), jnp.int32)
        dst_v[pl.ds(EW_PAD - 16, 16)] = jnp.zeros((16,), jnp.int32)
        w_v[pl.ds(EW_PAD - 16, 16)] = jnp.zeros((16,), jnp.float32)
        pltpu.sync_copy(src_hbm.at[pl.ds(ebase, EW)], src_v.at[pl.ds(0, EW)])
        pltpu.sync_copy(dst_hbm.at[pl.ds(ebase, EW)], dst_v.at[pl.ds(0, EW)])
        pltpu.sync_copy(w_hbm.at[pl.ds(ebase, EW)], w_v.at[pl.ds(0, EW)])

        def grp(g, _):
            off = g * 16
            s16 = src_v[pl.ds(off, 16)]
            d16 = dst_v[pl.ds(off, 16)]
            u16 = plsc.load_gather(u_v, [s16])
            v16 = plsc.load_gather(v_v, [d16])
            x = u16 + v16 + w_v[pl.ds(off, 16)]
            e16 = jnp.where(x >= 0.0, x, x * jnp.float32(0.01))
            t_v[pl.ds(off, 16)] = jnp.exp(e16)
            return 0
        lax.fori_loop(0, EW_PAD // 16, grp, 0)
        pltpu.sync_copy(t_v.at[pl.ds(0, EW)], t_out.at[pl.ds(ebase, EW)])

    one(u1_v, wss_ref, ss_src_ref, ss_dst_ref, tss_out)
    one(u2_v, wos_ref, os_src_ref, os_dst_ref, tos_out)


def _sc_t(u_ss, u_os, v_s, w_ss, w_os, ss_src, ss_dst, os_src, os_dst):
    mesh = plsc.VectorSubcoreMesh(core_axis_name="c", subcore_axis_name="s")
    f32 = jnp.float32
    run = pl.kernel(
        _sc_t_body,
        compiler_params=pltpu.CompilerParams(needs_layout_passes=False),
        out_type=[
            jax.ShapeDtypeStruct((E,), f32),
            jax.ShapeDtypeStruct((E,), f32),
        ],
        mesh=mesh,
        scratch_types=[
            pltpu.VMEM((N,), f32),
            pltpu.VMEM((N,), f32),
            pltpu.VMEM((N,), f32),
            pltpu.VMEM((EW_PAD,), jnp.int32),
            pltpu.VMEM((EW_PAD,), jnp.int32),
            pltpu.VMEM((EW_PAD,), f32),
            pltpu.VMEM((EW_PAD,), f32),
        ],
    )
    return run(u_ss, u_os, v_s, w_ss, w_os, ss_src, ss_dst, os_src, os_dst)


def _sc_body(tabss_ref, tabos_ref, tabo_ref, tss_ref, tos_ref,
             qssf_ref, qosf_ref,
             ss_src_ref, ss_dst_ref, os_src_ref, os_dst_ref,
             f_src_ref, f_dst_ref, b_src_ref, b_dst_ref, zp_ref,
             pss_out, qss_out, pos_out, qos_out, af_out, ab_out,
             p_sh, t_c, gidx_v, sidx_v, rows_v, t2_c, gidx2_v, sidx2_v,
             rows2_v, qf_v, sem, sem2):
    c = lax.axis_index("c")
    s = lax.axis_index("s")
    c_n = c * N
    ebase = s * EPT
    NT = N_PAD // NS
    ntail = EPT - (NCHUNK - 1) * CH

    def zero_p():
        rb0 = s * NT
        pltpu.sync_copy(zp_ref.at[pl.ds(rb0, NT)], p_sh.at[pl.ds(rb0, NT)])

    def stage_chunk(i, src_hbm, dst_hbm, t_hbm, gidx, sidx, t_b):
        goff = ebase + i * CH
        pltpu.sync_copy(src_hbm.at[pl.ds(goff, CH)], gidx)
        pltpu.sync_copy(dst_hbm.at[pl.ds(goff, CH)], sidx)
        if t_hbm is not None:
            pltpu.sync_copy(t_hbm.at[pl.ds(goff, CH)], t_b)

        def grp(g, _):
            off = g * 16
            gidx[pl.ds(off, 16)] = gidx[pl.ds(off, 16)] + c_n
            return 0
        lax.fori_loop(0, CH // 16, grp, 0)

        @pl.when(i == NCHUNK - 1)
        def _():
            def sent(g, _):
                sidx[pl.ds(ntail + g * 16, 16)] = \
                    jnp.full((16,), N, jnp.int32)
                return 0
            lax.fori_loop(0, (CH - ntail) // 16, sent, 0)

    def p_pass(tab_hbm, t_hbm, src_hbm, dst_hbm, p_out):
        zero_p()
        plsc.subcore_barrier()

        def consume(rows, t_b, sidx):
            if t_hbm is not None:
                def rowscale(j, _):
                    tb = plsc.load_gather(t_b, [jnp.full((16,), j, jnp.int32)])
                    for k in range(H // 16):
                        rows[j, pl.ds(k * 16, 16)] = \
                            rows[j, pl.ds(k * 16, 16)] * tb
                    return 0
                lax.fori_loop(0, CH, rowscale, 0)
            pltpu.sync_copy(rows, p_sh.at[sidx], add=True)

        stage_chunk(0, src_hbm, dst_hbm, t_hbm, gidx_v, sidx_v, t_c)
        pltpu.async_copy(tab_hbm.at[gidx_v], rows_v, sem)

        def pair(p, _):
            i = p * 2

            @pl.when(i + 1 < NCHUNK)
            def _():
                stage_chunk(i + 1, src_hbm, dst_hbm, t_hbm,
                            gidx2_v, sidx2_v, t2_c)
                pltpu.async_copy(tab_hbm.at[gidx2_v], rows2_v, sem2)

            pltpu.make_async_copy(tab_hbm.at[gidx_v], rows_v, sem).wait()
            consume(rows_v, t_c, sidx_v)

            @pl.when(i + 2 < NCHUNK)
            def _():
                stage_chunk(i + 2, src_hbm, dst_hbm, t_hbm,
                            gidx_v, sidx_v, t_c)
                pltpu.async_copy(tab_hbm.at[gidx_v], rows_v, sem)

            @pl.when(i + 1 < NCHUNK)
            def _():
                pltpu.make_async_copy(tab_hbm.at[gidx2_v], rows2_v,
                                      sem2).wait()
                consume(rows2_v, t2_c, sidx2_v)
            return 0
        lax.fori_loop(0, (NCHUNK + 1) // 2, pair, 0)
        plsc.subcore_barrier()
        pltpu.sync_copy(p_sh.at[pl.ds(s * NT, NT)],
                        p_out.at[c, pl.ds(s * NT, NT)])
        plsc.subcore_barrier()

    def q_chunks(qflat_hbm, t_hbm, src_hbm, dst_hbm):
        def chunk(i, _):
            stage_chunk(i, src_hbm, dst_hbm, t_hbm, gidx_v, sidx_v, t_c)
            goff = ebase + i * CH
            pltpu.sync_copy(qflat_hbm.at[pl.ds(goff * QW, CH * QW)], qf_v)

            def qscale(j, _):
                tb = plsc.load_gather(t_c, [jnp.full((16,), j, jnp.int32)])
                rows_v[j, pl.ds(0, 16)] = qf_v[pl.ds(j * QW, 16)] * tb
                return 0
            lax.fori_loop(0, CH, qscale, 0)
            pltpu.sync_copy(rows_v, p_sh.at[sidx_v], add=True)
            return 0
        lax.fori_loop(0, NCHUNK, chunk, 0)

    def q_pass():
        def zr(j, _):
            for k in range(H // 16):
                rows_v[j, pl.ds(k * 16, 16)] = jnp.zeros((16,), jnp.float32)
            return 0
        lax.fori_loop(0, CH, zr, 0)
        zero_p()
        plsc.subcore_barrier()

        @pl.when(c == 0)
        def _():
            q_chunks(qssf_ref, tss_ref, ss_src_ref, ss_dst_ref)

        @pl.when(c == 1)
        def _():
            q_chunks(qosf_ref, tos_ref, os_src_ref, os_dst_ref)
        plsc.subcore_barrier()

        @pl.when(c == 0)
        def _():
            pltpu.sync_copy(p_sh.at[pl.ds(s * NT, NT)],
                            qss_out.at[pl.ds(s * NT, NT)])

        @pl.when(c == 1)
        def _():
            pltpu.sync_copy(p_sh.at[pl.ds(s * NT, NT)],
                            qos_out.at[pl.ds(s * NT, NT)])
        plsc.subcore_barrier()

    p_pass(tabss_ref, tss_ref, ss_src_ref, ss_dst_ref, pss_out)
    p_pass(tabos_ref, tos_ref, os_src_ref, os_dst_ref, pos_out)
    q_pass()
    p_pass(tabo_ref, None, f_src_ref, f_dst_ref, af_out)
    p_pass(tabo_ref, None, b_src_ref, b_dst_ref, ab_out)


def _sc_agg(tab_ss, tab_os, tab_o, t_ss, t_os, qtab_ss, qtab_os,
            ss_src, ss_dst, os_src, os_dst, f_src, f_dst, b_src, b_dst):
    mesh = plsc.VectorSubcoreMesh(core_axis_name="c", subcore_axis_name="s")
    f32 = jnp.float32
    run = pl.kernel(
        _sc_body,
        compiler_params=pltpu.CompilerParams(needs_layout_passes=False),
        out_type=[
            jax.ShapeDtypeStruct((NC, N_PAD, H), f32),
            jax.ShapeDtypeStruct((N_PAD, H), f32),
            jax.ShapeDtypeStruct((NC, N_PAD, H), f32),
            jax.ShapeDtypeStruct((N_PAD, H), f32),
            jax.ShapeDtypeStruct((NC, N_PAD, H), f32),
            jax.ShapeDtypeStruct((NC, N_PAD, H), f32),
        ],
        mesh=mesh,
        scratch_types=[
            pltpu.VMEM_SHARED((N_PAD, H), f32),
            pltpu.VMEM((CH,), f32),
            pltpu.VMEM((CH,), jnp.int32),
            pltpu.VMEM((CH,), jnp.int32),
            pltpu.VMEM((CH, H), f32),
            pltpu.VMEM((CH,), f32),
            pltpu.VMEM((CH,), jnp.int32),
            pltpu.VMEM((CH,), jnp.int32),
            pltpu.VMEM((CH, H), f32),
            pltpu.VMEM((CH * QW,), f32),
            pltpu.SemaphoreType.DMA,
            pltpu.SemaphoreType.DMA,
        ],
    )
    zp = jnp.zeros((N_PAD, H), jnp.float32)
    return run(tab_ss.reshape(NC * N, H), tab_os.reshape(NC * N, H),
               tab_o.reshape(NC * N, H), t_ss, t_os,
               qtab_ss.reshape(-1), qtab_os.reshape(-1),
               ss_src, ss_dst, os_src, os_dst, f_src, f_dst, b_src, b_dst,
               zp)


_BLKO = 1000


def _post_body(pss_ref, qss_ref, pos_ref, qos_ref, af_ref, ab_ref, o_ref,
               wss2_ref, wssb_ref, wos2_ref, wosb_ref,
               win_w_ref, win_b_ref, wself_w_ref, wself_b_ref,
               wout_w_ref, wout_b_ref, wo_w_ref, wo_b_ref,
               z_ref, x_ref):
    def attn_half(p_ref, q_ref, w2_ref, b_ref, nf):
        p = jnp.concatenate([p_ref[0], p_ref[1]], axis=1)
        q = q_ref[...]
        den = q[:, nf:nf + 1]
        num = p + jnp.dot(q[:, :nf], w2_ref[...],
                          preferred_element_type=jnp.float32)
        return jnp.where(den > 0.0, num / den + b_ref[...][None, :], 0.0)

    z_ref[...] = (attn_half(pss_ref, qss_ref, wss2_ref, wssb_ref, 10)
                  + attn_half(pos_ref, qos_ref, wos2_ref, wosb_ref, 2))

    af = jnp.concatenate([af_ref[0], af_ref[1]], axis=1)
    ab = jnp.concatenate([ab_ref[0], ab_ref[1]], axis=1)
    o_blk = o_ref[...]
    h_in = jax.nn.relu(jnp.dot(af, win_w_ref[...],
                               preferred_element_type=jnp.float32)
                       + win_b_ref[...][None, :])
    h_self = jax.nn.relu(jnp.dot(o_blk, wself_w_ref[...],
                                 preferred_element_type=jnp.float32)
                         + wself_b_ref[...][None, :])
    h_out = jax.nn.relu(jnp.dot(ab, wout_w_ref[...],
                                preferred_element_type=jnp.float32)
                        + wout_b_ref[...][None, :])
    wo = wo_w_ref[...]
    x_ref[...] = (jnp.dot(h_in, wo[:D], preferred_element_type=jnp.float32)
                  + jnp.dot(h_self, wo[D:2 * D],
                            preferred_element_type=jnp.float32)
                  + jnp.dot(h_out, wo[2 * D:],
                            preferred_element_type=jnp.float32)
                  + wo_b_ref[...][None, :])


def _post(p_ss, q_ss, p_os, q_os, a_f, a_b, o_feat,
          wss2, wss_b, wos2, wos_b, win_w, win_b, wself_w, wself_b,
          wout_w, wout_b, wo_w, wo_b):
    nb = N // _BLKO

    def vec(d):
        return pl.BlockSpec((d,), lambda i: (0,))

    return pl.pallas_call(
        _post_body,
        grid=(nb,),
        in_specs=[
            pl.BlockSpec((NC, _BLKO, H), lambda i: (0, i, 0)),
            pl.BlockSpec((_BLKO, H), lambda i: (i, 0)),
            pl.BlockSpec((NC, _BLKO, H), lambda i: (0, i, 0)),
            pl.BlockSpec((_BLKO, H), lambda i: (i, 0)),
            pl.BlockSpec((NC, _BLKO, H), lambda i: (0, i, 0)),
            pl.BlockSpec((NC, _BLKO, H), lambda i: (0, i, 0)),
            pl.BlockSpec((_BLKO, D), lambda i: (i, 0)),
            pl.BlockSpec((10, D), lambda i: (0, 0)),
            vec(D),
            pl.BlockSpec((2, D), lambda i: (0, 0)),
            vec(D),
            pl.BlockSpec((D, D), lambda i: (0, 0)),
            vec(D),
            pl.BlockSpec((D, D), lambda i: (0, 0)),
            vec(D),
            pl.BlockSpec((D, D), lambda i: (0, 0)),
            vec(D),
            pl.BlockSpec((3 * D, D), lambda i: (0, 0)),
            vec(D),
        ],
        out_specs=[
            pl.BlockSpec((_BLKO, D), lambda i: (i, 0)),
            pl.BlockSpec((_BLKO, D), lambda i: (i, 0)),
        ],
        out_shape=[
            jax.ShapeDtypeStruct((N, D), jnp.float32),
            jax.ShapeDtypeStruct((N, D), jnp.float32),
        ],
    )(p_ss, q_ss, p_os, q_os, a_f, a_b, o_feat,
      wss2, wss_b, wos2, wos_b, win_w, win_b, wself_w, wself_b,
      wout_w, wout_b, wo_w, wo_b)


def kernel(s_feat, o_feat, ss_feat, os_feat, Ws_w, Ws_b, Wos_w, Wos_b,
           Wss_w, Wss_b, attn_w, attn_b, Win_w, Win_b, Wself_w, Wself_b,
           Wout_w, Wout_b, Wo_w, Wo_b,
           ss_edge_index, os_edge_index, fwd_edge_index, bwd_edge_index):
    a1 = attn_w[:D, 0]
    a2 = attn_w[D:, 0]

    tab_ss, tab_os, tab_o = _prep_nodes(s_feat, o_feat, Wss_w[:D], Wos_w[:D])
    u_ss, u_os, v_s = _prep_scalars(tab_ss, tab_os, s_feat, Ws_w, Ws_b,
                                    a1, a2)

    w_ss, w_os = _prep_edges(
        ss_feat, os_feat, Wss_w[D:], Wos_w[D:], Wss_b, Wos_b, a1, attn_b)

    ones = jnp.ones((E, 1), jnp.float32)
    pad_e = E_PAD - E
    qtab_ss = jnp.pad(
        jnp.concatenate([ss_feat, ones, jnp.zeros((E, QW - 11), jnp.float32)],
                        axis=1), ((0, pad_e), (0, 0)))
    qtab_os = jnp.pad(
        jnp.concatenate([os_feat, ones, jnp.zeros((E, QW - 3), jnp.float32)],
                        axis=1), ((0, pad_e), (0, 0)))

    idx = [jnp.pad(a.astype(jnp.int32), (0, CH)) for a in
           (ss_edge_index[0], ss_edge_index[1], os_edge_index[0],
            os_edge_index[1], fwd_edge_index[0], fwd_edge_index[1],
            bwd_edge_index[0], bwd_edge_index[1])]

    t_ss, t_os = _sc_t(u_ss, u_os, v_s, w_ss, w_os,
                       idx[0], idx[1], idx[2], idx[3])
    t_ss = jnp.pad(t_ss, (0, CH))
    t_os = jnp.pad(t_os, (0, CH))

    p_ss, q_ss, p_os, q_os, a_f, a_b = _sc_agg(
        tab_ss, tab_os, tab_o, t_ss, t_os, qtab_ss, qtab_os, *idx)

    z, x = _post(p_ss, q_ss, p_os, q_os, a_f, a_b, o_feat,
                 Wss_w[D:], Wss_b, Wos_w[D:], Wos_b, Win_w, Win_b,
                 Wself_w, Wself_b, Wout_w, Wout_b, Wo_w, Wo_b)
    return z, x


# speedup vs baseline: 3.7854x; 1.0103x over previous
import jax
import jax.numpy as jnp
from jax import lax
from jax.experimental import pallas as pl
from jax.experimental.pallas import tpu as pltpu
from jax.experimental.pallas import tpu_sc as plsc

N = 10000
E = 160000
D = 256
H = 128
NC = 2
NS = 16
EPT = E // NS
CH = 128
NCHUNK = -(-EPT // CH)
E_PAD = E + CH
EW = E // (NC * NS)
EW_PAD = EW + 16
N_PAD = 10240
QW = 16


_BLKN = 1000


def _prep_nodes_body(s_ref, o_ref, wss1_ref, wos1_ref,
                     tabss_ref, tabos_ref, tabo_ref):
    s_blk = s_ref[...]
    o_blk = o_ref[...]
    hss = jnp.dot(s_blk, wss1_ref[...], preferred_element_type=jnp.float32)
    hos = jnp.dot(o_blk, wos1_ref[...], preferred_element_type=jnp.float32)
    tabss_ref[...] = jnp.stack([hss[:, :H], hss[:, H:]])
    tabos_ref[...] = jnp.stack([hos[:, :H], hos[:, H:]])
    tabo_ref[...] = jnp.stack([o_blk[:, :H], o_blk[:, H:]])


def _prep_nodes(s_feat, o_feat, wss1, wos1):
    nb = N // _BLKN
    return pl.pallas_call(
        _prep_nodes_body,
        grid=(nb,),
        in_specs=[
            pl.BlockSpec((_BLKN, D), lambda i: (i, 0)),
            pl.BlockSpec((_BLKN, D), lambda i: (i, 0)),
            pl.BlockSpec((D, D), lambda i: (0, 0)),
            pl.BlockSpec((D, D), lambda i: (0, 0)),
        ],
        out_specs=[
            pl.BlockSpec((NC, _BLKN, H), lambda i: (0, i, 0)),
            pl.BlockSpec((NC, _BLKN, H), lambda i: (0, i, 0)),
            pl.BlockSpec((NC, _BLKN, H), lambda i: (0, i, 0)),
        ],
        out_shape=[
            jax.ShapeDtypeStruct((NC, N, H), jnp.float32),
            jax.ShapeDtypeStruct((NC, N, H), jnp.float32),
            jax.ShapeDtypeStruct((NC, N, H), jnp.float32),
        ],
    )(s_feat, o_feat, wss1, wos1)


def _prep_scalars_body(tabss_ref, tabos_ref, s_ref, wsw_ref, wsb_ref,
                       a1_ref, a2_ref, uss_ref, uos_ref, v_ref):
    a1_lo = a1_ref[pl.ds(0, H)][None, :]
    a1_hi = a1_ref[pl.ds(H, H)][None, :]
    uss_ref[...] = (jnp.sum(tabss_ref[0] * a1_lo, axis=1)
                    + jnp.sum(tabss_ref[1] * a1_hi, axis=1))
    uos_ref[...] = (jnp.sum(tabos_ref[0] * a1_lo, axis=1)
                    + jnp.sum(tabos_ref[1] * a1_hi, axis=1))
    a2 = a2_ref[...]
    m = jnp.sum(wsw_ref[...] * a2[None, :], axis=1)
    v_ref[...] = (jnp.sum(s_ref[...] * m[None, :], axis=1)
                  + jnp.sum(wsb_ref[...] * a2))


def _prep_scalars(tab_ss, tab_os, s_feat, ws_w, ws_b, a1, a2):
    return pl.pallas_call(
        _prep_scalars_body,
        out_shape=[
            jax.ShapeDtypeStruct((N,), jnp.float32),
            jax.ShapeDtypeStruct((N,), jnp.float32),
            jax.ShapeDtypeStruct((N,), jnp.float32),
        ],
    )(tab_ss, tab_os, s_feat, ws_w, ws_b, a1, a2)


_BLKE = 16000


def _prep_edges_body(ssf_ref, osf_ref, wss2_ref, wos2_ref, wssb_ref, wosb_ref,
                     a1_ref, ab_ref, wss_ref, wos_ref):
    a1 = a1_ref[...]
    g_ss = jnp.sum(wss2_ref[...] * a1[None, :], axis=1)
    g_os = jnp.sum(wos2_ref[...] * a1[None, :], axis=1)
    c_ss = jnp.sum(wssb_ref[...] * a1) + ab_ref[0]
    c_os = jnp.sum(wosb_ref[...] * a1) + ab_ref[0]
    sl = pl.ds(pl.program_id(0) * _BLKE, _BLKE)
    wss_ref[sl] = jnp.sum(ssf_ref[...] * g_ss[None, :], axis=1) + c_ss
    wos_ref[sl] = jnp.sum(osf_ref[...] * g_os[None, :], axis=1) + c_os


def _prep_edges(ss_feat, os_feat, wss2, wos2, wss_b, wos_b, a1, attn_b):
    nb = E // _BLKE
    return pl.pallas_call(
        _prep_edges_body,
        grid=(nb,),
        in_specs=[
            pl.BlockSpec((_BLKE, 10), lambda i: (i, 0)),
            pl.BlockSpec((_BLKE, 2), lambda i: (i, 0)),
            pl.BlockSpec((10, D), lambda i: (0, 0)),
            pl.BlockSpec((2, D), lambda i: (0, 0)),
            pl.BlockSpec((D,), lambda i: (0,)),
            pl.BlockSpec((D,), lambda i: (0,)),
            pl.BlockSpec((D,), lambda i: (0,)),
            pl.BlockSpec((1,), lambda i: (0,)),
        ],
        out_specs=[
            pl.BlockSpec((E,), lambda i: (0,)),
            pl.BlockSpec((E,), lambda i: (0,)),
        ],
        out_shape=[
            jax.ShapeDtypeStruct((E,), jnp.float32),
            jax.ShapeDtypeStruct((E,), jnp.float32),
        ],
    )(ss_feat, os_feat, wss2, wos2, wss_b, wos_b, a1, attn_b)


def _sc_t_body(uss_ref, uos_ref, v_ref, wss_ref, wos_ref,
               ss_src_ref, ss_dst_ref, os_src_ref, os_dst_ref,
               tss_out, tos_out,
               u1_v, u2_v, v_v, src_v, dst_v, w_v, t_v):
    c = lax.axis_index("c")
    s = lax.axis_index("s")
    wid = s * NC + c
    ebase = wid * EW

    pltpu.sync_copy(uss_ref, u1_v)
    pltpu.sync_copy(uos_ref, u2_v)
    pltpu.sync_copy(v_ref, v_v)

    def one(u_v, w_hbm, src_hbm, dst_hbm, t_out):
        src_v[pl.ds(EW_PAD - 16, 16)] = jnp.zeros((16,), jnp.int32)
        dst_v[pl.ds(EW_PAD - 16, 16)] = jnp.zeros((16,), jnp.int32)
        w_v[pl.ds(EW_PAD - 16, 16)] = jnp.zeros((16,), jnp.float32)
        pltpu.sync_copy(src_hbm.at[pl.ds(ebase, EW)], src_v.at[pl.ds(0, EW)])
        pltpu.sync_copy(dst_hbm.at[pl.ds(ebase, EW)], dst_v.at[pl.ds(0, EW)])
        pltpu.sync_copy(w_hbm.at[pl.ds(ebase, EW)], w_v.at[pl.ds(0, EW)])

        def grp(g, _):
            off = g * 16
            s16 = src_v[pl.ds(off, 16)]
            d16 = dst_v[pl.ds(off, 16)]
            u16 = plsc.load_gather(u_v, [s16])
            v16 = plsc.load_gather(v_v, [d16])
            x = u16 + v16 + w_v[pl.ds(off, 16)]
            e16 = jnp.where(x >= 0.0, x, x * jnp.float32(0.01))
            t_v[pl.ds(off, 16)] = jnp.exp(e16)
            return 0
        lax.fori_loop(0, EW_PAD // 16, grp, 0)
        pltpu.sync_copy(t_v.at[pl.ds(0, EW)], t_out.at[pl.ds(ebase, EW)])

    one(u1_v, wss_ref, ss_src_ref, ss_dst_ref, tss_out)
    one(u2_v, wos_ref, os_src_ref, os_dst_ref, tos_out)


def _sc_t(u_ss, u_os, v_s, w_ss, w_os, ss_src, ss_dst, os_src, os_dst):
    mesh = plsc.VectorSubcoreMesh(core_axis_name="c", subcore_axis_name="s")
    f32 = jnp.float32
    run = pl.kernel(
        _sc_t_body,
        compiler_params=pltpu.CompilerParams(needs_layout_passes=False),
        out_type=[
            jax.ShapeDtypeStruct((E,), f32),
            jax.ShapeDtypeStruct((E,), f32),
        ],
        mesh=mesh,
        scratch_types=[
            pltpu.VMEM((N,), f32),
            pltpu.VMEM((N,), f32),
            pltpu.VMEM((N,), f32),
            pltpu.VMEM((EW_PAD,), jnp.int32),
            pltpu.VMEM((EW_PAD,), jnp.int32),
            pltpu.VMEM((EW_PAD,), f32),
            pltpu.VMEM((EW_PAD,), f32),
        ],
    )
    return run(u_ss, u_os, v_s, w_ss, w_os, ss_src, ss_dst, os_src, os_dst)


def _sc_body(tabss_ref, tabos_ref, tabo_ref, tss_ref, tos_ref,
             qssf_ref, qosf_ref,
             ss_src_ref, ss_dst_ref, os_src_ref, os_dst_ref,
             f_src_ref, f_dst_ref, b_src_ref, b_dst_ref, zp_ref,
             pss_out, qss_out, pos_out, qos_out, af_out, ab_out,
             p_sh, t_c, gidx_v, sidx_v, rows_v, t2_c, gidx2_v, sidx2_v,
             rows2_v, qf_v, sem, sem2):
    c = lax.axis_index("c")
    s = lax.axis_index("s")
    c_n = c * N
    ebase = s * EPT
    NT = N_PAD // NS
    ntail = EPT - (NCHUNK - 1) * CH

    def zero_p():
        rb0 = s * NT
        pltpu.sync_copy(zp_ref.at[pl.ds(rb0, NT)], p_sh.at[pl.ds(rb0, NT)])

    def stage_chunk(i, src_hbm, dst_hbm, t_hbm, gidx, sidx, t_b):
        goff = ebase + i * CH
        pltpu.sync_copy(src_hbm.at[pl.ds(goff, CH)], gidx)
        pltpu.sync_copy(dst_hbm.at[pl.ds(goff, CH)], sidx)
        if t_hbm is not None:
            pltpu.sync_copy(t_hbm.at[pl.ds(goff, CH)], t_b)

        def grp(g, _):
            off = g * 16
            gidx[pl.ds(off, 16)] = gidx[pl.ds(off, 16)] + c_n
            return 0
        lax.fori_loop(0, CH // 16, grp, 0)

        @pl.when(i == NCHUNK - 1)
        def _():
            def sent(g, _):
                sidx[pl.ds(ntail + g * 16, 16)] = \
                    jnp.full((16,), N, jnp.int32)
                return 0
            lax.fori_loop(0, (CH - ntail) // 16, sent, 0)

    def p_pass(tab_hbm, t_hbm, src_hbm, dst_hbm, p_out):
        zero_p()
        plsc.subcore_barrier()

        def consume(rows, t_b, sidx):
            if t_hbm is not None:
                def rowscale(jj, _):
                    for u in range(4):
                        j = jj * 4 + u
                        tb = plsc.load_gather(
                            t_b, [jnp.full((16,), j, jnp.int32)])
                        for k in range(H // 16):
                            rows[j, pl.ds(k * 16, 16)] = \
                                rows[j, pl.ds(k * 16, 16)] * tb
                    return 0
                lax.fori_loop(0, CH // 4, rowscale, 0)
            pltpu.sync_copy(rows, p_sh.at[sidx], add=True)

        stage_chunk(0, src_hbm, dst_hbm, t_hbm, gidx_v, sidx_v, t_c)
        pltpu.async_copy(tab_hbm.at[gidx_v], rows_v, sem)

        def pair(p, _):
            i = p * 2

            @pl.when(i + 1 < NCHUNK)
            def _():
                stage_chunk(i + 1, src_hbm, dst_hbm, t_hbm,
                            gidx2_v, sidx2_v, t2_c)
                pltpu.async_copy(tab_hbm.at[gidx2_v], rows2_v, sem2)

            pltpu.make_async_copy(tab_hbm.at[gidx_v], rows_v, sem).wait()
            consume(rows_v, t_c, sidx_v)

            @pl.when(i + 2 < NCHUNK)
            def _():
                stage_chunk(i + 2, src_hbm, dst_hbm, t_hbm,
                            gidx_v, sidx_v, t_c)
                pltpu.async_copy(tab_hbm.at[gidx_v], rows_v, sem)

            @pl.when(i + 1 < NCHUNK)
            def _():
                pltpu.make_async_copy(tab_hbm.at[gidx2_v], rows2_v,
                                      sem2).wait()
                consume(rows2_v, t2_c, sidx2_v)
            return 0
        lax.fori_loop(0, (NCHUNK + 1) // 2, pair, 0)
        plsc.subcore_barrier()
        pltpu.sync_copy(p_sh.at[pl.ds(s * NT, NT)],
                        p_out.at[c, pl.ds(s * NT, NT)])
        plsc.subcore_barrier()

    def q_chunks(qflat_hbm, t_hbm, src_hbm, dst_hbm):
        def chunk(i, _):
            stage_chunk(i, src_hbm, dst_hbm, t_hbm, gidx_v, sidx_v, t_c)
            goff = ebase + i * CH
            pltpu.sync_copy(qflat_hbm.at[pl.ds(goff * QW, CH * QW)], qf_v)

            def qscale(j, _):
                tb = plsc.load_gather(t_c, [jnp.full((16,), j, jnp.int32)])
                rows_v[j, pl.ds(0, 16)] = qf_v[pl.ds(j * QW, 16)] * tb
                return 0
            lax.fori_loop(0, CH, qscale, 0)
            pltpu.sync_copy(rows_v, p_sh.at[sidx_v], add=True)
            return 0
        lax.fori_loop(0, NCHUNK, chunk, 0)

    def q_pass():
        def zr(j, _):
            for k in range(H // 16):
                rows_v[j, pl.ds(k * 16, 16)] = jnp.zeros((16,), jnp.float32)
            return 0
        lax.fori_loop(0, CH, zr, 0)
        zero_p()
        plsc.subcore_barrier()

        @pl.when(c == 0)
        def _():
            q_chunks(qssf_ref, tss_ref, ss_src_ref, ss_dst_ref)

        @pl.when(c == 1)
        def _():
            q_chunks(qosf_ref, tos_ref, os_src_ref, os_dst_ref)
        plsc.subcore_barrier()

        @pl.when(c == 0)
        def _():
            pltpu.sync_copy(p_sh.at[pl.ds(s * NT, NT)],
                            qss_out.at[pl.ds(s * NT, NT)])

        @pl.when(c == 1)
        def _():
            pltpu.sync_copy(p_sh.at[pl.ds(s * NT, NT)],
                            qos_out.at[pl.ds(s * NT, NT)])
        plsc.subcore_barrier()

    p_pass(tabss_ref, tss_ref, ss_src_ref, ss_dst_ref, pss_out)
    p_pass(tabos_ref, tos_ref, os_src_ref, os_dst_ref, pos_out)
    q_pass()
    p_pass(tabo_ref, None, f_src_ref, f_dst_ref, af_out)
    p_pass(tabo_ref, None, b_src_ref, b_dst_ref, ab_out)


def _sc_agg(tab_ss, tab_os, tab_o, t_ss, t_os, qtab_ss, qtab_os,
            ss_src, ss_dst, os_src, os_dst, f_src, f_dst, b_src, b_dst):
    mesh = plsc.VectorSubcoreMesh(core_axis_name="c", subcore_axis_name="s")
    f32 = jnp.float32
    run = pl.kernel(
        _sc_body,
        compiler_params=pltpu.CompilerParams(needs_layout_passes=False),
        out_type=[
            jax.ShapeDtypeStruct((NC, N_PAD, H), f32),
            jax.ShapeDtypeStruct((N_PAD, H), f32),
            jax.ShapeDtypeStruct((NC, N_PAD, H), f32),
            jax.ShapeDtypeStruct((N_PAD, H), f32),
            jax.ShapeDtypeStruct((NC, N_PAD, H), f32),
            jax.ShapeDtypeStruct((NC, N_PAD, H), f32),
        ],
        mesh=mesh,
        scratch_types=[
            pltpu.VMEM_SHARED((N_PAD, H), f32),
            pltpu.VMEM((CH,), f32),
            pltpu.VMEM((CH,), jnp.int32),
            pltpu.VMEM((CH,), jnp.int32),
            pltpu.VMEM((CH, H), f32),
            pltpu.VMEM((CH,), f32),
            pltpu.VMEM((CH,), jnp.int32),
            pltpu.VMEM((CH,), jnp.int32),
            pltpu.VMEM((CH, H), f32),
            pltpu.VMEM((CH * QW,), f32),
            pltpu.SemaphoreType.DMA,
            pltpu.SemaphoreType.DMA,
        ],
    )
    zp = jnp.zeros((N_PAD, H), jnp.float32)
    return run(tab_ss.reshape(NC * N, H), tab_os.reshape(NC * N, H),
               tab_o.reshape(NC * N, H), t_ss, t_os,
               qtab_ss.reshape(-1), qtab_os.reshape(-1),
               ss_src, ss_dst, os_src, os_dst, f_src, f_dst, b_src, b_dst,
               zp)


_BLKO = 1000


def _post_body(pss_ref, qss_ref, pos_ref, qos_ref, af_ref, ab_ref, o_ref,
               wss2_ref, wssb_ref, wos2_ref, wosb_ref,
               win_w_ref, win_b_ref, wself_w_ref, wself_b_ref,
               wout_w_ref, wout_b_ref, wo_w_ref, wo_b_ref,
               z_ref, x_ref):
    def attn_half(p_ref, q_ref, w2_ref, b_ref, nf):
        p = jnp.concatenate([p_ref[0], p_ref[1]], axis=1)
        q = q_ref[...]
        den = q[:, nf:nf + 1]
        num = p + jnp.dot(q[:, :nf], w2_ref[...],
                          preferred_element_type=jnp.float32)
        return jnp.where(den > 0.0, num / den + b_ref[...][None, :], 0.0)

    z_ref[...] = (attn_half(pss_ref, qss_ref, wss2_ref, wssb_ref, 10)
                  + attn_half(pos_ref, qos_ref, wos2_ref, wosb_ref, 2))

    af = jnp.concatenate([af_ref[0], af_ref[1]], axis=1)
    ab = jnp.concatenate([ab_ref[0], ab_ref[1]], axis=1)
    o_blk = o_ref[...]
    h_in = jax.nn.relu(jnp.dot(af, win_w_ref[...],
                               preferred_element_type=jnp.float32)
                       + win_b_ref[...][None, :])
    h_self = jax.nn.relu(jnp.dot(o_blk, wself_w_ref[...],
                                 preferred_element_type=jnp.float32)
                         + wself_b_ref[...][None, :])
    h_out = jax.nn.relu(jnp.dot(ab, wout_w_ref[...],
                                preferred_element_type=jnp.float32)
                        + wout_b_ref[...][None, :])
    wo = wo_w_ref[...]
    x_ref[...] = (jnp.dot(h_in, wo[:D], preferred_element_type=jnp.float32)
                  + jnp.dot(h_self, wo[D:2 * D],
                            preferred_element_type=jnp.float32)
                  + jnp.dot(h_out, wo[2 * D:],
                            preferred_element_type=jnp.float32)
                  + wo_b_ref[...][None, :])


def _post(p_ss, q_ss, p_os, q_os, a_f, a_b, o_feat,
          wss2, wss_b, wos2, wos_b, win_w, win_b, wself_w, wself_b,
          wout_w, wout_b, wo_w, wo_b):
    nb = N // _BLKO

    def vec(d):
        return pl.BlockSpec((d,), lambda i: (0,))

    return pl.pallas_call(
        _post_body,
        grid=(nb,),
        in_specs=[
            pl.BlockSpec((NC, _BLKO, H), lambda i: (0, i, 0)),
            pl.BlockSpec((_BLKO, H), lambda i: (i, 0)),
            pl.BlockSpec((NC, _BLKO, H), lambda i: (0, i, 0)),
            pl.BlockSpec((_BLKO, H), lambda i: (i, 0)),
            pl.BlockSpec((NC, _BLKO, H), lambda i: (0, i, 0)),
            pl.BlockSpec((NC, _BLKO, H), lambda i: (0, i, 0)),
            pl.BlockSpec((_BLKO, D), lambda i: (i, 0)),
            pl.BlockSpec((10, D), lambda i: (0, 0)),
            vec(D),
            pl.BlockSpec((2, D), lambda i: (0, 0)),
            vec(D),
            pl.BlockSpec((D, D), lambda i: (0, 0)),
            vec(D),
            pl.BlockSpec((D, D), lambda i: (0, 0)),
            vec(D),
            pl.BlockSpec((D, D), lambda i: (0, 0)),
            vec(D),
            pl.BlockSpec((3 * D, D), lambda i: (0, 0)),
            vec(D),
        ],
        out_specs=[
            pl.BlockSpec((_BLKO, D), lambda i: (i, 0)),
            pl.BlockSpec((_BLKO, D), lambda i: (i, 0)),
        ],
        out_shape=[
            jax.ShapeDtypeStruct((N, D), jnp.float32),
            jax.ShapeDtypeStruct((N, D), jnp.float32),
        ],
    )(p_ss, q_ss, p_os, q_os, a_f, a_b, o_feat,
      wss2, wss_b, wos2, wos_b, win_w, win_b, wself_w, wself_b,
      wout_w, wout_b, wo_w, wo_b)


def kernel(s_feat, o_feat, ss_feat, os_feat, Ws_w, Ws_b, Wos_w, Wos_b,
           Wss_w, Wss_b, attn_w, attn_b, Win_w, Win_b, Wself_w, Wself_b,
           Wout_w, Wout_b, Wo_w, Wo_b,
           ss_edge_index, os_edge_index, fwd_edge_index, bwd_edge_index):
    a1 = attn_w[:D, 0]
    a2 = attn_w[D:, 0]

    tab_ss, tab_os, tab_o = _prep_nodes(s_feat, o_feat, Wss_w[:D], Wos_w[:D])
    u_ss, u_os, v_s = _prep_scalars(tab_ss, tab_os, s_feat, Ws_w, Ws_b,
                                    a1, a2)

    w_ss, w_os = _prep_edges(
        ss_feat, os_feat, Wss_w[D:], Wos_w[D:], Wss_b, Wos_b, a1, attn_b)

    ones = jnp.ones((E, 1), jnp.float32)
    pad_e = E_PAD - E
    qtab_ss = jnp.pad(
        jnp.concatenate([ss_feat, ones, jnp.zeros((E, QW - 11), jnp.float32)],
                        axis=1), ((0, pad_e), (0, 0)))
    qtab_os = jnp.pad(
        jnp.concatenate([os_feat, ones, jnp.zeros((E, QW - 3), jnp.float32)],
                        axis=1), ((0, pad_e), (0, 0)))

    idx = [jnp.pad(a.astype(jnp.int32), (0, CH)) for a in
           (ss_edge_index[0], ss_edge_index[1], os_edge_index[0],
            os_edge_index[1], fwd_edge_index[0], fwd_edge_index[1],
            bwd_edge_index[0], bwd_edge_index[1])]

    t_ss, t_os = _sc_t(u_ss, u_os, v_s, w_ss, w_os,
                       idx[0], idx[1], idx[2], idx[3])
    t_ss = jnp.pad(t_ss, (0, CH))
    t_os = jnp.pad(t_os, (0, CH))

    p_ss, q_ss, p_os, q_os, a_f, a_b = _sc_agg(
        tab_ss, tab_os, tab_o, t_ss, t_os, qtab_ss, qtab_os, *idx)

    z, x = _post(p_ss, q_ss, p_os, q_os, a_f, a_b, o_feat,
                 Wss_w[D:], Wss_b, Wos_w[D:], Wos_b, Win_w, Win_b,
                 Wself_w, Wself_b, Wout_w, Wout_b, Wo_w, Wo_b)
    return z, x
